# baseline (speedup 1.0000x reference)
import jax
import jax.numpy as jnp
import numpy as np
from jax.experimental import pallas as pl
from jax.experimental.pallas import tpu as pltpu

B, NK, NF, D = 4096, 8000, 1024, 64
BB = 512
HB = BB // 2
RK = 512
NCH = 16
LAST = NK - (NCH - 1) * RK
NBUF = 8
T = B // BB

_CHUNKS = [(j * RK, RK if j < NCH - 1 else LAST) for j in range(NCH)]


def _dma(prot_hbm, bufs, sems, j, blk):
    off, sz = _CHUNKS[j]
    b = j % NBUF
    return pltpu.make_async_copy(
        prot_hbm.at[pl.ds(off, sz), pl.ds(blk * BB, BB)],
        bufs.at[b, pl.ds(0, sz), :],
        sems.at[b])


def _fused_kernel(prot_hbm, comp_ref, Wp_ref, bp_ref, Wc_ref, bc_ref,
                  Wb_ref, bb_ref, Wl_ref, bl_ref, S_ref, out_ref,
                  bufs, acc_ref, sems):
    t = pl.program_id(0)

    @pl.when(t == 0)
    def _():
        for j in range(NBUF):
            _dma(prot_hbm, bufs, sems, j, 0).start()

    for j in range(NCH):
        off, sz = _CHUNKS[j]
        b = j % NBUF
        _dma(prot_hbm, bufs, sems, j, t).wait()
        part = jnp.dot(Wp_ref[:, off:off + sz], bufs[b, 0:sz, :],
                       preferred_element_type=jnp.float32)
        if j == 0:
            acc_ref[...] = part
        else:
            acc_ref[...] = acc_ref[...] + part
        if j + NBUF < NCH:
            _dma(prot_hbm, bufs, sems, j + NBUF, t).start()
        else:
            nxt = j + NBUF - NCH

            @pl.when(t + 1 < T)
            def _():
                _dma(prot_hbm, bufs, sems, nxt, t + 1).start()

    def bvec(ref, n):
        return jax.lax.broadcast_in_dim(ref[...], (D, n), (0,))

    pT = jnp.maximum(acc_ref[...] + bvec(bp_ref, BB), 0.0)
    for h in range(2):
        col0 = h * HB
        cT = jax.lax.dot_general(Wc_ref[...], comp_ref[col0:col0 + HB, :],
                                 (((1,), (1,)), ((), ())),
                                 preferred_element_type=jnp.float32)
        cT = jnp.maximum(cT + bvec(bc_ref, HB), 0.0)
        mT = jnp.dot(Wb_ref[...], cT, preferred_element_type=jnp.float32)
        Z = mT * jnp.tile(pT[:, col0:col0 + HB], (D, 1))
        joinedT = jnp.dot(S_ref[...], Z, preferred_element_type=jnp.float32)
        joinedT = jnp.maximum(joinedT + bvec(bb_ref, HB), 0.0)
        out_ref[:, col0:col0 + HB] = (
            jnp.dot(Wl_ref[...], joinedT, preferred_element_type=jnp.float32)
            + bl_ref[...])


def kernel(protein_input, compound_input, Wp, bp, Wc, bc, Wb, bb, Wl, bl):
    protT = protein_input.T
    WpT = Wp.T
    WcT = Wc.T
    Wb_r = Wb.reshape(D * D, D)
    WlT = Wl.T
    S = jnp.asarray(np.kron(np.eye(D, dtype=np.float32),
                            np.ones((1, D), dtype=np.float32)))
    grid = (T,)
    outT = pl.pallas_call(
        _fused_kernel,
        grid=grid,
        in_specs=[
            pl.BlockSpec(memory_space=pltpu.MemorySpace.HBM),
            pl.BlockSpec((BB, NF), lambda i: (i, 0)),
            pl.BlockSpec((D, NK), lambda i: (0, 0)),
            pl.BlockSpec((D,), lambda i: (0,)),
            pl.BlockSpec((D, NF), lambda i: (0, 0)),
            pl.BlockSpec((D,), lambda i: (0,)),
            pl.BlockSpec((D * D, D), lambda i: (0, 0)),
            pl.BlockSpec((D,), lambda i: (0,)),
            pl.BlockSpec((1, D), lambda i: (0, 0)),
            pl.BlockSpec((1, 1), lambda i: (0, 0)),
            pl.BlockSpec((D, D * D), lambda i: (0, 0)),
        ],
        out_specs=pl.BlockSpec((1, BB), lambda i: (0, i)),
        out_shape=jax.ShapeDtypeStruct((1, B), jnp.float32),
        scratch_shapes=[
            pltpu.VMEM((NBUF, RK, BB), jnp.float32),
            pltpu.VMEM((D, BB), jnp.float32),
            pltpu.SemaphoreType.DMA((NBUF,)),
        ],
        )(protT, compound_input, WpT, bp, WcT, bc, Wb_r, bb, WlT,
          bl.reshape(1, 1), S)
    return outT.reshape(B, 1)


# speedup vs baseline: 1.0754x; 1.0754x over previous
import jax
import jax.numpy as jnp
import numpy as np
from jax.experimental import pallas as pl
from jax.experimental.pallas import tpu as pltpu

B, NK, NF, D = 4096, 8000, 1024, 64
BB = 256
NC = 40
RK = NK // NC
T = B // (2 * BB)


def _bcast_col(vec, n):
    return jax.lax.broadcast_in_dim(vec, (D, n), (0,))


def _dma(prot_hbm, buf, sem, blk):
    col0 = blk * BB
    return [
        pltpu.make_async_copy(
            prot_hbm.at[pl.ds(j * RK, RK), pl.ds(col0, BB)],
            buf.at[pl.ds(j * RK, RK), :],
            sem.at[j])
        for j in range(NC)
    ]


def _fused_kernel(prot_hbm, comp_ref, Wp_ref, bp_ref, Wc_ref, bc_ref,
                  Wb_ref, bb_ref, Wl_ref, bl_ref, S_ref, out_ref,
                  bufA, bufB, semA, semB):
    t = pl.program_id(0)

    @pl.when(t == 0)
    def _():
        for cp in _dma(prot_hbm, bufA, semA, 0):
            cp.start()
        for cp in _dma(prot_hbm, bufB, semB, 1):
            cp.start()

    def half(buf, sem, blk, col0):
        for cp in _dma(prot_hbm, buf, sem, blk):
            cp.wait()
        pT = jnp.dot(Wp_ref[...], buf[...],
                     preferred_element_type=jnp.float32)
        @pl.when(blk + 2 < 2 * T)
        def _():
            for cp in _dma(prot_hbm, buf, sem, blk + 2):
                cp.start()
        pT = jnp.maximum(pT + _bcast_col(bp_ref[...], BB), 0.0)
        cT = jax.lax.dot_general(Wc_ref[...], comp_ref[pl.ds(col0, BB), :],
                                 (((1,), (1,)), ((), ())),
                                 preferred_element_type=jnp.float32)
        cT = jnp.maximum(cT + _bcast_col(bc_ref[...], BB), 0.0)
        mT = jnp.dot(Wb_ref[...], cT, preferred_element_type=jnp.float32)
        Z = mT * jnp.tile(pT, (D, 1))
        joinedT = jnp.dot(S_ref[...], Z, preferred_element_type=jnp.float32)
        joinedT = jnp.maximum(joinedT + _bcast_col(bb_ref[...], BB), 0.0)
        out_ref[:, pl.ds(col0, BB)] = (
            jnp.dot(Wl_ref[...], joinedT, preferred_element_type=jnp.float32)
            + bl_ref[...])

    half(bufA, semA, 2 * t, 0)
    half(bufB, semB, 2 * t + 1, BB)


def kernel(protein_input, compound_input, Wp, bp, Wc, bc, Wb, bb, Wl, bl):
    protT = protein_input.T
    WpT = Wp.T
    WcT = Wc.T
    Wb_r = Wb.reshape(D * D, D)
    WlT = Wl.T
    S = jnp.asarray(np.kron(np.eye(D, dtype=np.float32),
                            np.ones((1, D), dtype=np.float32)))
    grid = (T,)
    outT = pl.pallas_call(
        _fused_kernel,
        grid=grid,
        in_specs=[
            pl.BlockSpec(memory_space=pltpu.MemorySpace.HBM),
            pl.BlockSpec((2 * BB, NF), lambda i: (i, 0)),
            pl.BlockSpec((D, NK), lambda i: (0, 0)),
            pl.BlockSpec((D,), lambda i: (0,)),
            pl.BlockSpec((D, NF), lambda i: (0, 0)),
            pl.BlockSpec((D,), lambda i: (0,)),
            pl.BlockSpec((D * D, D), lambda i: (0, 0)),
            pl.BlockSpec((D,), lambda i: (0,)),
            pl.BlockSpec((1, D), lambda i: (0, 0)),
            pl.BlockSpec((1, 1), lambda i: (0, 0)),
            pl.BlockSpec((D, D * D), lambda i: (0, 0)),
        ],
        out_specs=pl.BlockSpec((1, 2 * BB), lambda i: (0, i)),
        out_shape=jax.ShapeDtypeStruct((1, B), jnp.float32),
        scratch_shapes=[
            pltpu.VMEM((NK, BB), jnp.float32),
            pltpu.VMEM((NK, BB), jnp.float32),
            pltpu.SemaphoreType.DMA((NC,)),
            pltpu.SemaphoreType.DMA((NC,)),
        ],
        )(protT, compound_input, WpT, bp, WcT, bc, Wb_r, bb, WlT,
          bl.reshape(1, 1), S)
    return outT.reshape(B, 1)


# speedup vs baseline: 1.2578x; 1.1697x over previous
import jax
import jax.numpy as jnp
import numpy as np
from jax.experimental import pallas as pl
from jax.experimental.pallas import tpu as pltpu

B, NK, NF, D = 4096, 8000, 1024, 64
BB = 256
NC = 25
RK = NK // NC
T = B // (2 * BB)


def _bcast_col(vec, n):
    return jax.lax.broadcast_in_dim(vec, (D, n), (0,))


def _dma(prot_hbm, buf, sem, blk):
    col0 = blk * BB
    return [
        pltpu.make_async_copy(
            prot_hbm.at[pl.ds(j * RK, RK), pl.ds(col0, BB)],
            buf.at[pl.ds(j * RK, RK), :],
            sem.at[j])
        for j in range(NC)
    ]


def _fused_kernel(prot_hbm, comp_ref, Wp_ref, bp_ref, Wc_ref, bc_ref,
                  Wb_ref, bb_ref, Wl_ref, bl_ref, S_ref, out_ref,
                  bufA, bufB, semA, semB):
    t = pl.program_id(0)

    @pl.when(t == 0)
    def _():
        for cp in _dma(prot_hbm, bufA, semA, 0):
            cp.start()
        for cp in _dma(prot_hbm, bufB, semB, 1):
            cp.start()

    def half(buf, sem, blk, col0):
        for cp in _dma(prot_hbm, buf, sem, blk):
            cp.wait()
        pT = jnp.dot(Wp_ref[...], buf[...],
                     preferred_element_type=jnp.float32)
        @pl.when(blk + 2 < 2 * T)
        def _():
            for cp in _dma(prot_hbm, buf, sem, blk + 2):
                cp.start()
        pT = jnp.maximum(pT + _bcast_col(bp_ref[...], BB), 0.0)
        cT = jax.lax.dot_general(Wc_ref[...], comp_ref[pl.ds(col0, BB), :],
                                 (((1,), (1,)), ((), ())),
                                 preferred_element_type=jnp.float32)
        cT = jnp.maximum(cT + _bcast_col(bc_ref[...], BB), 0.0)
        mT = jnp.dot(Wb_ref[...], cT, preferred_element_type=jnp.float32)
        Z = mT * jnp.tile(pT, (D, 1))
        joinedT = jnp.dot(S_ref[...], Z, preferred_element_type=jnp.float32)
        joinedT = jnp.maximum(joinedT + _bcast_col(bb_ref[...], BB), 0.0)
        out_ref[:, pl.ds(col0, BB)] = (
            jnp.dot(Wl_ref[...], joinedT, preferred_element_type=jnp.float32)
            + bl_ref[...])

    half(bufA, semA, 2 * t, 0)
    half(bufB, semB, 2 * t + 1, BB)


def kernel(protein_input, compound_input, Wp, bp, Wc, bc, Wb, bb, Wl, bl):
    protT = protein_input.T
    WpT = Wp.T
    WcT = Wc.T
    Wb_r = Wb.reshape(D * D, D)
    WlT = Wl.T
    S = jnp.asarray(np.kron(np.eye(D, dtype=np.float32),
                            np.ones((1, D), dtype=np.float32)))
    grid = (T,)
    outT = pl.pallas_call(
        _fused_kernel,
        grid=grid,
        in_specs=[
            pl.BlockSpec(memory_space=pltpu.MemorySpace.HBM),
            pl.BlockSpec((2 * BB, NF), lambda i: (i, 0)),
            pl.BlockSpec((D, NK), lambda i: (0, 0)),
            pl.BlockSpec((D,), lambda i: (0,)),
            pl.BlockSpec((D, NF), lambda i: (0, 0)),
            pl.BlockSpec((D,), lambda i: (0,)),
            pl.BlockSpec((D * D, D), lambda i: (0, 0)),
            pl.BlockSpec((D,), lambda i: (0,)),
            pl.BlockSpec((1, D), lambda i: (0, 0)),
            pl.BlockSpec((1, 1), lambda i: (0, 0)),
            pl.BlockSpec((D, D * D), lambda i: (0, 0)),
        ],
        out_specs=pl.BlockSpec((1, 2 * BB), lambda i: (0, i)),
        out_shape=jax.ShapeDtypeStruct((1, B), jnp.float32),
        scratch_shapes=[
            pltpu.VMEM((NK, BB), jnp.float32),
            pltpu.VMEM((NK, BB), jnp.float32),
            pltpu.SemaphoreType.DMA((NC,)),
            pltpu.SemaphoreType.DMA((NC,)),
        ],
        )(protT, compound_input, WpT, bp, WcT, bc, Wb_r, bb, WlT,
          bl.reshape(1, 1), S)
    return outT.reshape(B, 1)
